# baseline (speedup 1.0000x reference)
import jax
import jax.numpy as jnp
from jax.experimental import pallas as pl
from jax.experimental.pallas import tpu as pltpu


def _round_up(x, m):
    return ((x + m - 1) // m) * m


def _mlp_kernel(xt_ref, waug_ref, w1_ref, b1_ref, w2h_ref, b2_ref,
                w25_ref, b25_ref, w3_ref, b3_ref,
                w4_ref, b4_ref, w5_ref, b5_ref, out_ref):
    bf16 = jnp.bfloat16
    f32 = jnp.float32
    xt = xt_ref[...]
    w1 = w1_ref[...]
    b1 = b1_ref[...]
    K = xt.shape[0]
    tb = xt.shape[1]
    km = K // 2

    xm = xt[:km]
    xmh = xm.astype(bf16)
    xml = (xm - xmh.astype(f32)).astype(bf16)
    ones = jnp.ones((2, tb), bf16)
    xta = jnp.concatenate([xmh, xml, xmh, ones], axis=0)
    h1a = jnp.maximum(
        jnp.dot(waug_ref[...], xta,
                preferred_element_type=f32).astype(bf16), 0)

    slabs = [
        jnp.maximum((w1 * xt[k:k + 1, :] + b1).astype(bf16), 0)
        for k in range(km, K)
    ]
    h1 = jnp.concatenate([h1a] + slabs, axis=0)

    y = jnp.dot(w2h_ref[...], h1, preferred_element_type=f32)
    y = jnp.maximum(y + b2_ref[...], 0.0)
    y = jnp.maximum(
        jnp.dot(w25_ref[...], y, preferred_element_type=f32)
        + b25_ref[...], 0.0)
    y = jnp.maximum(
        jnp.dot(w3_ref[...], y, preferred_element_type=f32)
        + b3_ref[...], 0.0)
    y = jnp.maximum(
        jnp.dot(w4_ref[...], y, preferred_element_type=f32)
        + b4_ref[...], 0.0)
    y = (jnp.dot(w5_ref[...], y, preferred_element_type=f32)
         + b5_ref[...])
    out_ref[...] = y[None].astype(out_ref.dtype)


def kernel(x, w1, b1, w2, b2, w25, b25, w3, b3, w4, b4, w5, b5):
    B, K = x.shape
    H = w1.shape[0]
    f32 = jnp.float32
    bf16 = jnp.bfloat16

    xt = x.T

    lane = 128
    tb = min(4096, _round_up(B, lane))
    padded_b = _round_up(B, tb)
    if padded_b // tb < 2 and padded_b > lane:
        tb = _round_up(pl.cdiv(padded_b, 2), lane)
        padded_b = tb * pl.cdiv(padded_b, tb)
    if padded_b != B:
        xt = jnp.pad(xt, ((0, 0), (0, padded_b - B)))
    grid = (padded_b // tb,)

    km = K // 2
    w1blk = jnp.kron(jnp.eye(km, dtype=f32), w1)
    w1hi = w1blk.astype(bf16)
    w1lo = (w1blk - w1hi.astype(f32)).astype(bf16)
    b1col = jnp.tile(b1.reshape(H, 1), (km, 1))
    b1hi = b1col.astype(bf16)
    b1lo = (b1col - b1hi.astype(f32)).astype(bf16)
    waug = jnp.concatenate([w1hi, w1hi, w1lo, b1hi, b1lo],
                           axis=1)

    w2hi = w2.astype(bf16)

    def col(v):
        return v.reshape(-1, 1)

    args = (xt, waug, w1.reshape(H, 1), col(b1), w2hi, col(b2),
            w25, col(b25), w3, col(b3), w4, col(b4), w5, col(b5))

    in_specs = [pl.BlockSpec((K, tb), lambda i: (0, i))]
    in_specs += [pl.BlockSpec(a.shape, lambda i: (0, 0))
                 for a in args[1:]]

    out = pl.pallas_call(
        _mlp_kernel,
        out_shape=jax.ShapeDtypeStruct((grid[0], 1, tb), x.dtype),
        grid=grid,
        in_specs=in_specs,
        out_specs=pl.BlockSpec((1, 1, tb), lambda i: (i, 0, 0)),
        compiler_params=pltpu.CompilerParams(
            dimension_semantics=("arbitrary",),
            vmem_limit_bytes=64 * 1024 * 1024),
    )(*args)
    return out.reshape(-1)[:B].reshape(B, 1)


# speedup vs baseline: 1.3986x; 1.3986x over previous
import jax
import jax.numpy as jnp
from jax.experimental import pallas as pl
from jax.experimental.pallas import tpu as pltpu


def _round_up(x, m):
    return ((x + m - 1) // m) * m


def _mlp_kernel(xt_ref, w1_ref, b1_ref, w2h_ref, b2_ref,
                w25_ref, b25_ref, w3_ref, b3_ref,
                w4_ref, b4_ref, w5_ref, b5_ref, out_ref):
    bf16 = jnp.bfloat16
    f32 = jnp.float32
    xt = xt_ref[...]
    w1 = w1_ref[...]
    b1 = b1_ref[...]
    K = xt.shape[0]

    slabs = [
        jnp.maximum((w1 * xt[k:k + 1, :] + b1).astype(bf16), 0)
        for k in range(K)
    ]
    h1 = jnp.concatenate(slabs, axis=0)

    y = jnp.dot(w2h_ref[...], h1, preferred_element_type=f32)
    y = jnp.maximum(y + b2_ref[...], 0.0)
    y = jnp.maximum(
        jnp.dot(w25_ref[...], y, preferred_element_type=f32)
        + b25_ref[...], 0.0)
    y = jnp.maximum(
        jnp.dot(w3_ref[...], y, preferred_element_type=f32)
        + b3_ref[...], 0.0)
    y = jnp.maximum(
        jnp.dot(w4_ref[...], y, preferred_element_type=f32)
        + b4_ref[...], 0.0)
    y = (jnp.dot(w5_ref[...], y, preferred_element_type=f32)
         + b5_ref[...])
    out_ref[...] = y[None].astype(out_ref.dtype)


def kernel(x, w1, b1, w2, b2, w25, b25, w3, b3, w4, b4, w5, b5):
    B, K = x.shape
    H = w1.shape[0]
    f32 = jnp.float32
    bf16 = jnp.bfloat16

    xt = x.T

    lane = 128
    tb = min(8192, _round_up(B, lane))
    padded_b = _round_up(B, tb)
    if padded_b // tb < 2 and padded_b > lane:
        tb = _round_up(pl.cdiv(padded_b, 2), lane)
        padded_b = tb * pl.cdiv(padded_b, tb)
    if padded_b != B:
        xt = jnp.pad(xt, ((0, 0), (0, padded_b - B)))
    grid = (padded_b // tb,)

    w2hi = w2.astype(bf16)

    def col(v):
        return v.reshape(-1, 1)

    args = (xt, w1.reshape(H, 1), col(b1), w2hi, col(b2),
            w25, col(b25), w3, col(b3), w4, col(b4), w5, col(b5))

    in_specs = [pl.BlockSpec((K, tb), lambda i: (0, i))]
    in_specs += [pl.BlockSpec(a.shape, lambda i: (0, 0))
                 for a in args[1:]]

    out = pl.pallas_call(
        _mlp_kernel,
        out_shape=jax.ShapeDtypeStruct((grid[0], 1, tb), x.dtype),
        grid=grid,
        in_specs=in_specs,
        out_specs=pl.BlockSpec((1, 1, tb), lambda i: (i, 0, 0)),
        compiler_params=pltpu.CompilerParams(
            dimension_semantics=("arbitrary",),
            vmem_limit_bytes=64 * 1024 * 1024),
    )(*args)
    return out.reshape(-1)[:B].reshape(B, 1)


# speedup vs baseline: 1.5122x; 1.0812x over previous
import jax
import jax.numpy as jnp
from jax.experimental import pallas as pl
from jax.experimental.pallas import tpu as pltpu


def _round_up(x, m):
    return ((x + m - 1) // m) * m


def _mlp_kernel(xt_ref, w1_ref, b1_ref, w2h_ref, b2_ref,
                w25_ref, b25_ref, w3_ref, b3_ref,
                w4_ref, b4_ref, w5_ref, b5_ref, out_ref):
    bf16 = jnp.bfloat16
    f32 = jnp.float32
    xt = xt_ref[...]
    w1 = w1_ref[...]
    b1 = b1_ref[...]
    K = xt.shape[0]

    slabs = [
        jnp.maximum((w1 * xt[k:k + 1, :] + b1).astype(bf16), 0)
        for k in range(K)
    ]
    h1 = jnp.concatenate(slabs, axis=0)

    y = jnp.dot(w2h_ref[...], h1, preferred_element_type=f32)
    y = jnp.maximum(y + b2_ref[...], 0.0)
    y = jnp.maximum(
        jnp.dot(w25_ref[...], y, preferred_element_type=f32)
        + b25_ref[...], 0.0)
    y = jnp.maximum(
        jnp.dot(w3_ref[...], y, preferred_element_type=f32)
        + b3_ref[...], 0.0)
    y = jnp.maximum(
        jnp.dot(w4_ref[...], y, preferred_element_type=f32)
        + b4_ref[...], 0.0)
    y = (jnp.dot(w5_ref[...], y, preferred_element_type=f32)
         + b5_ref[...])
    out_ref[...] = y[None].astype(out_ref.dtype)


def kernel(x, w1, b1, w2, b2, w25, b25, w3, b3, w4, b4, w5, b5):
    B, K = x.shape
    H = w1.shape[0]
    f32 = jnp.float32
    bf16 = jnp.bfloat16

    xt = x.T

    lane = 128
    tb = min(16384, _round_up(B, lane))
    padded_b = _round_up(B, tb)
    if padded_b // tb < 2 and padded_b > lane:
        tb = _round_up(pl.cdiv(padded_b, 2), lane)
        padded_b = tb * pl.cdiv(padded_b, tb)
    if padded_b != B:
        xt = jnp.pad(xt, ((0, 0), (0, padded_b - B)))
    grid = (padded_b // tb,)

    w2hi = w2.astype(bf16)

    def col(v):
        return v.reshape(-1, 1)

    args = (xt, w1.reshape(H, 1), col(b1), w2hi, col(b2),
            w25, col(b25), w3, col(b3), w4, col(b4), w5, col(b5))

    in_specs = [pl.BlockSpec((K, tb), lambda i: (0, i))]
    in_specs += [pl.BlockSpec(a.shape, lambda i: (0, 0))
                 for a in args[1:]]

    out = pl.pallas_call(
        _mlp_kernel,
        out_shape=jax.ShapeDtypeStruct((grid[0], 1, tb), x.dtype),
        grid=grid,
        in_specs=in_specs,
        out_specs=pl.BlockSpec((1, 1, tb), lambda i: (i, 0, 0)),
        compiler_params=pltpu.CompilerParams(
            dimension_semantics=("arbitrary",),
            vmem_limit_bytes=64 * 1024 * 1024),
    )(*args)
    return out.reshape(-1)[:B].reshape(B, 1)
